# baseline (speedup 1.0000x reference)
import jax
import jax.numpy as jnp
from jax import lax
from jax.experimental import pallas as pl
from jax.experimental.pallas import tpu as pltpu
from jax.experimental.pallas import tpu_sc as plsc

_NUM_CLASSES = 100000
_FEAT = 64
_BATCH = 16384
_NC = 2
_NS = 16
_NW = _NC * _NS
_BPW = _BATCH // _NW
_CHUNK = 128
_NCHUNK = _BPW // _CHUNK


def _cl_kernel(feat_hbm, lab_hbm, cent_hbm, out_hbm,
               lab_v, rows_v, feat_v, acc_v, gsem, fsem):
    wid = lax.axis_index("c") * _NS + lax.axis_index("s")
    base = wid * _BPW

    pltpu.sync_copy(lab_hbm.at[pl.ds(wid * _NCHUNK, _NCHUNK)], lab_v)

    def stage(j):
        buf = rows_v.at[j % 2]

        def issue(g, _):
            labv = lab_v[j, pl.ds(g * 16, 16)]
            for lane in range(16):
                l = labv[lane]
                pltpu.async_copy(cent_hbm.at[pl.ds(l, 1)],
                                 buf.at[pl.ds(g * 16 + lane, 1)],
                                 gsem.at[j % 2])
            return 0

        lax.fori_loop(0, _CHUNK // 16, issue, 0)
        fc = pltpu.async_copy(feat_hbm.at[pl.ds(base + j * _CHUNK, _CHUNK)],
                              feat_v.at[j % 2], fsem.at[j % 2])
        return fc

    def drain(j):
        pltpu.make_async_copy(cent_hbm.at[pl.ds(0, _CHUNK)],
                              rows_v.at[j % 2], gsem.at[j % 2]).wait()

    zeros = jnp.zeros((16,), jnp.float32)
    accs = (zeros, zeros, zeros, zeros)
    pend = stage(0)
    for j in range(_NCHUNK):
        pend.wait()
        drain(j)
        if j + 1 < _NCHUNK:
            pend = stage(j + 1)
        rows = rows_v.at[j % 2]
        feat = feat_v.at[j % 2]

        def row_body(r, accs, _rows=rows, _feat=feat):
            a0, a1, a2, a3 = accs
            f0 = _feat[r, pl.ds(0, 16)]
            c0 = _rows[r, pl.ds(0, 16)]
            d0 = f0 - c0
            a0 = a0 + d0 * d0
            f1 = _feat[r, pl.ds(16, 16)]
            c1 = _rows[r, pl.ds(16, 16)]
            d1 = f1 - c1
            a1 = a1 + d1 * d1
            f2 = _feat[r, pl.ds(32, 16)]
            c2 = _rows[r, pl.ds(32, 16)]
            d2 = f2 - c2
            a2 = a2 + d2 * d2
            f3 = _feat[r, pl.ds(48, 16)]
            c3 = _rows[r, pl.ds(48, 16)]
            d3 = f3 - c3
            a3 = a3 + d3 * d3
            return (a0, a1, a2, a3)

        accs = lax.fori_loop(0, _CHUNK, row_body, accs)

    total = (accs[0] + accs[1]) + (accs[2] + accs[3])
    acc_v[...] = total * jnp.float32(1.0 / _BATCH)
    pltpu.sync_copy(acc_v, out_hbm.at[wid])


def _transpose_tc_kernel(ct_ref, out_ref):
    eye = jnp.eye(_FEAT, dtype=jnp.float32)
    out_ref[...] = lax.dot_general(
        ct_ref[...], eye, (((0,), (0,)), ((), ())),
        preferred_element_type=jnp.float32)


_TBLK = 512
_TGRID = (_NUM_CLASSES + _TBLK - 1) // _TBLK


@jax.jit
def _center_loss(features, labels, centers):
    labels2 = labels.reshape(_BATCH // _CHUNK, _CHUNK)
    centers_rm = pl.pallas_call(
        _transpose_tc_kernel,
        out_shape=jax.ShapeDtypeStruct((_NUM_CLASSES, _FEAT), jnp.float32),
        grid=(_TGRID,),
        in_specs=[pl.BlockSpec((_FEAT, _TBLK), lambda i: (0, i))],
        out_specs=pl.BlockSpec((_TBLK, _FEAT), lambda i: (i, 0)),
    )(centers.T)
    mesh = plsc.VectorSubcoreMesh(
        core_axis_name="c", subcore_axis_name="s",
        num_cores=_NC, num_subcores=_NS)
    out = pl.kernel(
        _cl_kernel,
        out_type=jax.ShapeDtypeStruct((_NW, 16), jnp.float32),
        mesh=mesh,
        scratch_types=[
            pltpu.VMEM((_NCHUNK, _CHUNK), jnp.int32),
            pltpu.VMEM((2, _CHUNK, _FEAT), jnp.float32),
            pltpu.VMEM((2, _CHUNK, _FEAT), jnp.float32),
            pltpu.VMEM((16,), jnp.float32),
            pltpu.SemaphoreType.DMA((2,)),
            pltpu.SemaphoreType.DMA((2,)),
        ],
    )(features, labels2, centers_rm)
    return jnp.sum(out)


def kernel(features, labels, centers):
    return _center_loss(features, labels.astype(jnp.int32), centers)


# speedup vs baseline: 2.3536x; 2.3536x over previous
import jax
import jax.numpy as jnp
from jax import lax
from jax.experimental import pallas as pl
from jax.experimental.pallas import tpu as pltpu
from jax.experimental.pallas import tpu_sc as plsc

_NUM_CLASSES = 100000
_FEAT = 64
_BATCH = 16384
_NC = 2
_NS = 16
_NW = _NC * _NS
_BPW = _BATCH // _NW
_CHUNK = 128
_NCHUNK = _BPW // _CHUNK


def _cl_kernel(feat_hbm, lab_hbm, cent_hbm, out_hbm,
               lab_v, rows_v, feat_v, acc_v, gsem, fsem):
    wid = lax.axis_index("c") * _NS + lax.axis_index("s")
    base = wid * _BPW

    pltpu.sync_copy(lab_hbm.at[pl.ds(wid * _NCHUNK, _NCHUNK)], lab_v)

    def stage(j):
        buf = rows_v.at[j % 2]

        def issue(g, _):
            labv = lab_v[j, pl.ds(g * 16, 16)]
            for lane in range(16):
                l = labv[lane]
                pltpu.async_copy(cent_hbm.at[pl.ds(l, 1)],
                                 buf.at[pl.ds(g * 16 + lane, 1)],
                                 gsem.at[j % 2])
            return 0

        lax.fori_loop(0, _CHUNK // 16, issue, 0)
        fc = pltpu.async_copy(feat_hbm.at[pl.ds(base + j * _CHUNK, _CHUNK)],
                              feat_v.at[j % 2], fsem.at[j % 2])
        return fc

    def drain(j):
        pltpu.make_async_copy(cent_hbm.at[pl.ds(0, _CHUNK)],
                              rows_v.at[j % 2], gsem.at[j % 2]).wait()

    zeros = jnp.zeros((16,), jnp.float32)
    accs = (zeros, zeros, zeros, zeros)
    pend = stage(0)
    for j in range(_NCHUNK):
        pend.wait()
        drain(j)
        if j + 1 < _NCHUNK:
            pend = stage(j + 1)
        rows = rows_v.at[j % 2]
        feat = feat_v.at[j % 2]

        def row_body(r, accs, _rows=rows, _feat=feat):
            a0, a1, a2, a3 = accs
            f0 = _feat[r, pl.ds(0, 16)]
            c0 = _rows[r, pl.ds(0, 16)]
            d0 = f0 - c0
            a0 = a0 + d0 * d0
            f1 = _feat[r, pl.ds(16, 16)]
            c1 = _rows[r, pl.ds(16, 16)]
            d1 = f1 - c1
            a1 = a1 + d1 * d1
            f2 = _feat[r, pl.ds(32, 16)]
            c2 = _rows[r, pl.ds(32, 16)]
            d2 = f2 - c2
            a2 = a2 + d2 * d2
            f3 = _feat[r, pl.ds(48, 16)]
            c3 = _rows[r, pl.ds(48, 16)]
            d3 = f3 - c3
            a3 = a3 + d3 * d3
            return (a0, a1, a2, a3)

        accs = lax.fori_loop(0, _CHUNK, row_body, accs)

    total = (accs[0] + accs[1]) + (accs[2] + accs[3])
    acc_v[...] = total * jnp.float32(1.0 / _BATCH)
    pltpu.sync_copy(acc_v, out_hbm.at[wid])


@jax.jit
def _center_loss(features, labels, centers):
    labels2 = labels.reshape(_BATCH // _CHUNK, _CHUNK)
    mesh = plsc.VectorSubcoreMesh(
        core_axis_name="c", subcore_axis_name="s",
        num_cores=_NC, num_subcores=_NS)
    out = pl.kernel(
        _cl_kernel,
        out_type=jax.ShapeDtypeStruct((_NW, 16), jnp.float32),
        mesh=mesh,
        scratch_types=[
            pltpu.VMEM((_NCHUNK, _CHUNK), jnp.int32),
            pltpu.VMEM((2, _CHUNK, _FEAT), jnp.float32),
            pltpu.VMEM((2, _CHUNK, _FEAT), jnp.float32),
            pltpu.VMEM((16,), jnp.float32),
            pltpu.SemaphoreType.DMA((2,)),
            pltpu.SemaphoreType.DMA((2,)),
        ],
    )(features, labels2, centers)
    return jnp.sum(out)


def kernel(features, labels, centers):
    return _center_loss(features, labels.astype(jnp.int32), centers)
